# baseline (speedup 1.0000x reference)
import functools

import jax
import jax.numpy as jnp
from jax import lax
from jax.experimental import pallas as pl
from jax.experimental.pallas import tpu as pltpu
from jax.experimental.pallas import tpu_sc as plsc

B, S, D_MODEL = 1, 2048, 768
RANK = 32
N_COMPRESS = 64
TOP_K = 2

BLK = 2048
NEG = -1e30
NW = 32
TPW = S // NW
L = 16
NCH = TPW // L


def _tc_scores_body(x_ref, wr_ref, scores_ref):
    scores_ref[...] = lax.dot_general(
        x_ref[...], wr_ref[...],
        dimension_numbers=(((1,), (1,)), ((), ())),
        preferred_element_type=jnp.float32)


def _tc_scores(x2d, W_router):
    return pl.pallas_call(
        _tc_scores_body,
        grid=(S // BLK,),
        in_specs=[
            pl.BlockSpec((BLK, D_MODEL), lambda i: (i, 0)),
            pl.BlockSpec((N_COMPRESS, D_MODEL), lambda i: (0, 0)),
        ],
        out_specs=pl.BlockSpec((BLK, N_COMPRESS), lambda i: (i, 0)),
        out_shape=jax.ShapeDtypeStruct((S, N_COMPRESS), jnp.float32),
    )(x2d, W_router)


def _sc_body(scores_hbm, mask_hbm, w_hbm, idx_hbm,
             score_v, mask_v, w_v, ti_v):
    wid = lax.axis_index("s") * 2 + lax.axis_index("c")
    base = wid * TPW
    pltpu.sync_copy(scores_hbm.at[pl.ds(base, TPW)], score_v)

    lanes = lax.iota(jnp.int32, L)
    zero_f = jnp.zeros((L,), jnp.float32)
    zero_i = jnp.zeros((L,), jnp.int32)
    toks = [c * L + lanes for c in range(NCH)]

    for t in range(TPW):
        for q in range(N_COMPRESS // L):
            mask_v[t, pl.ds(q * L, L)] = zero_f

    init = tuple((zero_f + NEG, zero_i, zero_f + NEG, zero_i)
                 for _ in range(NCH))

    def scan_body(nb, carry):
        st = [list(s) for s in carry]
        for j in range(4):
            col = nb * 4 + j + zero_i
            for c in range(NCH):
                m1, i1, m2, i2 = st[c]
                v = plsc.load_gather(score_v, [toks[c], col])
                gt1 = v > m1
                gt2 = jnp.logical_and(jnp.logical_not(gt1), v > m2)
                st[c] = [
                    jnp.where(gt1, v, m1),
                    jnp.where(gt1, col, i1),
                    jnp.where(gt1, m1, jnp.where(gt2, v, m2)),
                    jnp.where(gt1, i1, jnp.where(gt2, col, i2)),
                ]
        return tuple(tuple(s) for s in st)

    state = lax.fori_loop(0, N_COMPRESS // 4, scan_body, init)

    for c in range(NCH):
        m1, i1, m2, i2 = state[c]
        tok = toks[c]
        e = jnp.exp(m2 - m1)
        w1 = 1.0 / (1.0 + e)
        w2 = 1.0 - w1
        plsc.store_scatter(w_v, [tok, zero_i], w1)
        plsc.store_scatter(w_v, [tok, zero_i + 1], w2)
        plsc.store_scatter(ti_v, [tok, zero_i], i1)
        plsc.store_scatter(ti_v, [tok, zero_i + 1], i2)
        plsc.store_scatter(mask_v, [tok, i1], w1)
        plsc.store_scatter(mask_v, [tok, i2], w2)

    pltpu.sync_copy(mask_v, mask_hbm.at[pl.ds(base, TPW)])
    pltpu.sync_copy(w_v, w_hbm.at[0, pl.ds(base, TPW)])
    pltpu.sync_copy(ti_v, idx_hbm.at[0, pl.ds(base, TPW)])


def _sc_stage(scores):
    mesh = plsc.VectorSubcoreMesh(core_axis_name="c", subcore_axis_name="s")
    run = pl.kernel(
        _sc_body,
        mesh=mesh,
        out_type=[
            jax.ShapeDtypeStruct((S, N_COMPRESS), jnp.float32),
            jax.ShapeDtypeStruct((B, S, TOP_K), jnp.float32),
            jax.ShapeDtypeStruct((B, S, TOP_K), jnp.int32),
        ],
        scratch_types=[
            pltpu.VMEM((TPW, N_COMPRESS), jnp.float32),
            pltpu.VMEM((TPW, N_COMPRESS), jnp.float32),
            pltpu.VMEM((TPW, TOP_K), jnp.float32),
            pltpu.VMEM((TPW, TOP_K), jnp.int32),
        ],
        compiler_params=pltpu.CompilerParams(needs_layout_passes=False),
    )
    return run(scores)


def _tc_proj_body(x_ref, wf_ref, mask_ref, out_ref):
    proj = jnp.dot(x_ref[...], wf_ref[...],
                   preferred_element_type=jnp.float32)
    row_n = lax.broadcasted_iota(jnp.int32, (N_COMPRESS, N_COMPRESS * RANK), 0)
    col_n = lax.broadcasted_iota(jnp.int32, (N_COMPRESS, N_COMPRESS * RANK),
                                 1) // RANK
    expand = (row_n == col_n).astype(jnp.float32)
    mask_exp = jnp.dot(mask_ref[...], expand,
                       preferred_element_type=jnp.float32)
    row = lax.broadcasted_iota(jnp.int32, (N_COMPRESS * RANK, RANK), 0) % RANK
    col = lax.broadcasted_iota(jnp.int32, (N_COMPRESS * RANK, RANK), 1)
    gather_eye = (row == col).astype(jnp.float32)
    out_ref[...] = jnp.dot(proj * mask_exp, gather_eye,
                           preferred_element_type=jnp.float32)


def _tc_proj(x2d, wf, mask):
    return pl.pallas_call(
        _tc_proj_body,
        grid=(S // BLK,),
        in_specs=[
            pl.BlockSpec((BLK, D_MODEL), lambda i: (i, 0)),
            pl.BlockSpec((D_MODEL, N_COMPRESS * RANK), lambda i: (0, 0)),
            pl.BlockSpec((BLK, N_COMPRESS), lambda i: (i, 0)),
        ],
        out_specs=pl.BlockSpec((BLK, RANK), lambda i: (i, 0)),
        out_shape=jax.ShapeDtypeStruct((S, RANK), jnp.float32),
    )(x2d, wf, mask)


@jax.jit
def kernel(x, W_router, compress_neurons):
    x2d = x.reshape(S, D_MODEL)
    wf = compress_neurons.transpose(1, 0, 2).reshape(D_MODEL,
                                                     N_COMPRESS * RANK)
    scores = _tc_scores(x2d, W_router)
    mask, w, idx = _sc_stage(scores)
    out = _tc_proj(x2d, wf, mask)
    return (out.reshape(B, S, RANK), w, idx)


# speedup vs baseline: 1.0052x; 1.0052x over previous
import jax
import jax.numpy as jnp
from jax import lax
from jax.experimental import pallas as pl
from jax.experimental.pallas import tpu as pltpu
from jax.experimental.pallas import tpu_sc as plsc

B, S, D_MODEL = 1, 2048, 768
RANK = 32
N_COMPRESS = 64
TOP_K = 2

BLK = 1024
NEG = -1e30
NW = 32
TPW = S // NW
L = 16
NCH = TPW // L


def _tc_scores_body(x_ref, wr_ref, scores_ref):
    scores_ref[...] = lax.dot_general(
        x_ref[...], wr_ref[...],
        dimension_numbers=(((1,), (1,)), ((), ())),
        preferred_element_type=jnp.float32)


def _tc_scores(x2d, W_router):
    return pl.pallas_call(
        _tc_scores_body,
        grid=(S // BLK,),
        in_specs=[
            pl.BlockSpec((BLK, D_MODEL), lambda i: (i, 0)),
            pl.BlockSpec((N_COMPRESS, D_MODEL), lambda i: (0, 0)),
        ],
        out_specs=pl.BlockSpec((BLK, N_COMPRESS), lambda i: (i, 0)),
        out_shape=jax.ShapeDtypeStruct((S, N_COMPRESS), jnp.float32),
    )(x2d, W_router)


def _sc_body(scores_hbm, mask_hbm, w_hbm, idx_hbm,
             score_v, mask_v, w_v, ti_v):
    wid = lax.axis_index("s") * 2 + lax.axis_index("c")
    base = wid * TPW
    pltpu.sync_copy(scores_hbm.at[pl.ds(base, TPW)], score_v)

    lanes = lax.iota(jnp.int32, L)
    zero_f = jnp.zeros((L,), jnp.float32)
    zero_i = jnp.zeros((L,), jnp.int32)
    toks = [c * L + lanes for c in range(NCH)]

    for t in range(TPW):
        for q in range(N_COMPRESS // L):
            mask_v[t, pl.ds(q * L, L)] = zero_f

    init = tuple((zero_f + NEG, zero_i, zero_f + NEG, zero_i)
                 for _ in range(NCH))

    def scan_body(nb, carry):
        st = [list(s) for s in carry]
        for j in range(4):
            col = nb * 4 + j + zero_i
            for c in range(NCH):
                m1, i1, m2, i2 = st[c]
                v = plsc.load_gather(score_v, [toks[c], col])
                gt1 = v > m1
                gt2 = jnp.logical_and(jnp.logical_not(gt1), v > m2)
                st[c] = [
                    jnp.where(gt1, v, m1),
                    jnp.where(gt1, col, i1),
                    jnp.where(gt1, m1, jnp.where(gt2, v, m2)),
                    jnp.where(gt1, i1, jnp.where(gt2, col, i2)),
                ]
        return tuple(tuple(s) for s in st)

    state = lax.fori_loop(0, N_COMPRESS // 4, scan_body, init)

    for c in range(NCH):
        m1, i1, m2, i2 = state[c]
        tok = toks[c]
        e = jnp.exp(m2 - m1)
        w1 = 1.0 / (1.0 + e)
        w2 = 1.0 - w1
        plsc.store_scatter(w_v, [tok, zero_i], w1)
        plsc.store_scatter(w_v, [tok, zero_i + 1], w2)
        plsc.store_scatter(ti_v, [tok, zero_i], i1)
        plsc.store_scatter(ti_v, [tok, zero_i + 1], i2)
        plsc.store_scatter(mask_v, [tok, i1], w1)
        plsc.store_scatter(mask_v, [tok, i2], w2)

    pltpu.sync_copy(mask_v, mask_hbm.at[pl.ds(base, TPW)])
    pltpu.sync_copy(w_v, w_hbm.at[0, pl.ds(base, TPW)])
    pltpu.sync_copy(ti_v, idx_hbm.at[0, pl.ds(base, TPW)])


def _sc_stage(scores):
    mesh = plsc.VectorSubcoreMesh(core_axis_name="c", subcore_axis_name="s")
    run = pl.kernel(
        _sc_body,
        mesh=mesh,
        out_type=[
            jax.ShapeDtypeStruct((S, N_COMPRESS), jnp.float32),
            jax.ShapeDtypeStruct((B, S, TOP_K), jnp.float32),
            jax.ShapeDtypeStruct((B, S, TOP_K), jnp.int32),
        ],
        scratch_types=[
            pltpu.VMEM((TPW, N_COMPRESS), jnp.float32),
            pltpu.VMEM((TPW, N_COMPRESS), jnp.float32),
            pltpu.VMEM((TPW, TOP_K), jnp.float32),
            pltpu.VMEM((TPW, TOP_K), jnp.int32),
        ],
        compiler_params=pltpu.CompilerParams(needs_layout_passes=False),
    )
    return run(scores)


def _tc_proj_body(x_ref, wf_ref, mask_ref, out_ref):
    proj = jnp.dot(x_ref[...], wf_ref[...],
                   preferred_element_type=jnp.float32)
    row_n = lax.broadcasted_iota(jnp.int32, (N_COMPRESS, N_COMPRESS * RANK), 0)
    col_n = lax.broadcasted_iota(jnp.int32, (N_COMPRESS, N_COMPRESS * RANK),
                                 1) // RANK
    expand = (row_n == col_n).astype(jnp.float32)
    mask_exp = jnp.dot(mask_ref[...], expand,
                       preferred_element_type=jnp.float32)
    row = lax.broadcasted_iota(jnp.int32, (N_COMPRESS * RANK, RANK), 0) % RANK
    col = lax.broadcasted_iota(jnp.int32, (N_COMPRESS * RANK, RANK), 1)
    gather_eye = (row == col).astype(jnp.float32)
    out_ref[...] = jnp.dot(proj * mask_exp, gather_eye,
                           preferred_element_type=jnp.float32)


def _tc_proj(x2d, wf, mask):
    return pl.pallas_call(
        _tc_proj_body,
        grid=(S // BLK,),
        in_specs=[
            pl.BlockSpec((BLK, D_MODEL), lambda i: (i, 0)),
            pl.BlockSpec((D_MODEL, N_COMPRESS * RANK), lambda i: (0, 0)),
            pl.BlockSpec((BLK, N_COMPRESS), lambda i: (i, 0)),
        ],
        out_specs=pl.BlockSpec((BLK, RANK), lambda i: (i, 0)),
        out_shape=jax.ShapeDtypeStruct((S, RANK), jnp.float32),
    )(x2d, wf, mask)


@jax.jit
def kernel(x, W_router, compress_neurons):
    x2d = x.reshape(S, D_MODEL)
    wf = compress_neurons.transpose(1, 0, 2).reshape(D_MODEL,
                                                     N_COMPRESS * RANK)
    scores = _tc_scores(x2d, W_router)
    mask, w, idx = _sc_stage(scores)
    out = _tc_proj(x2d, wf, mask)
    return (out.reshape(B, S, RANK), w, idx)
